# Initial kernel scaffold; baseline (speedup 1.0000x reference)
import jax
import jax.numpy as jnp
from jax.experimental import pallas as pl


def kernel(x, edge_index, W1, b1, W2, b2, W3, b3):
    raise NotImplementedError("write your pallas kernel here")


# speedup vs baseline: 10.5795x; 10.5795x over previous
import functools

import jax
import jax.numpy as jnp
from jax import lax
from jax.experimental import pallas as pl
from jax.experimental.pallas import tpu as pltpu
from jax.experimental.pallas import tpu_sc as plsc

_NC = 2
_NS = 16
_NW = _NC * _NS
_K = 128
_BR = 2000


def _mesh():
    return plsc.VectorSubcoreMesh(
        core_axis_name="c", subcore_axis_name="s",
        num_cores=_NC, num_subcores=_NS)


def _deg_kernel(n_acc, chunks):
    rpw = n_acc // _NS

    @functools.partial(
        pl.kernel,
        out_type=jax.ShapeDtypeStruct((_NC, n_acc, 128), jnp.float32),
        mesh=_mesh(),
        scratch_types=[
            pltpu.VMEM((chunks, _K), jnp.int32),
            pltpu.VMEM((_K, 128), jnp.float32),
            pltpu.VMEM_SHARED((n_acc, 128), jnp.float32),
        ],
    )
    def deg(dsti_hbm, out_hbm, dstv, buf, acc):
        cid = lax.axis_index("c")
        sid = lax.axis_index("s")
        wid = cid * _NS + sid
        pltpu.sync_copy(dsti_hbm.at[wid], dstv)

        @pl.loop(0, _K)
        def _(r):
            for cc in range(8):
                buf.at[pl.ds(r, 1), pl.ds(cc * 16, 16)][...] = (
                    jnp.zeros((1, 16), jnp.float32))

        row0 = sid * rpw

        @pl.loop(0, rpw // _K)
        def _(j):
            pltpu.sync_copy(buf, acc.at[pl.ds(row0 + j * _K, _K)])

        @pl.loop(0, _K)
        def _(r):
            buf.at[pl.ds(r, 1), pl.ds(0, 16)][...] = jnp.ones((1, 16), jnp.float32)

        plsc.subcore_barrier()

        @pl.loop(0, chunks)
        def _(j):
            pltpu.sync_copy(buf, acc.at[dstv.at[j]], add=True)

        plsc.subcore_barrier()

        @pl.loop(0, rpw // _K)
        def _(j):
            sl = pl.ds(row0 + j * _K, _K)
            pltpu.sync_copy(acc.at[sl], out_hbm.at[cid, sl])

    return deg


def _agg_kernel(n_acc, d, chunks):
    rpw = n_acc // _NS

    @functools.partial(
        pl.kernel,
        out_type=jax.ShapeDtypeStruct((_NC, n_acc, d), jnp.float32),
        mesh=_mesh(),
        scratch_types=[
            pltpu.VMEM((chunks, _K), jnp.int32),
            pltpu.VMEM((chunks, _K), jnp.int32),
            pltpu.VMEM((_K, d), jnp.float32),
            pltpu.VMEM_SHARED((n_acc, d), jnp.float32),
        ],
    )
    def agg(g_hbm, srci_hbm, dsti_hbm, out_hbm, srcv, dstv, rows, acc):
        cid = lax.axis_index("c")
        sid = lax.axis_index("s")
        wid = cid * _NS + sid
        pltpu.sync_copy(srci_hbm.at[wid], srcv)
        pltpu.sync_copy(dsti_hbm.at[wid], dstv)

        @pl.loop(0, _K)
        def _(r):
            for cc in range(d // 16):
                rows.at[pl.ds(r, 1), pl.ds(cc * 16, 16)][...] = (
                    jnp.zeros((1, 16), jnp.float32))

        row0 = sid * rpw

        @pl.loop(0, rpw // _K)
        def _(j):
            pltpu.sync_copy(rows, acc.at[pl.ds(row0 + j * _K, _K)])

        plsc.subcore_barrier()

        @pl.loop(0, chunks)
        def _(j):
            pltpu.sync_copy(g_hbm.at[srcv.at[j]], rows)
            pltpu.sync_copy(rows, acc.at[dstv.at[j]], add=True)

        plsc.subcore_barrier()

        @pl.loop(0, rpw // _K)
        def _(j):
            sl = pl.ds(row0 + j * _K, _K)
            pltpu.sync_copy(acc.at[sl], out_hbm.at[cid, sl])

    return agg


def _first_tc(n, n_acc, d_in, hid):
    grid = n // _BR

    def body(deg_ref, x_ref, w_ref, dinv_ref, g_ref):
        deg = deg_ref[0, :, :16] + deg_ref[1, :, :16]
        total = jnp.sum(deg, axis=1, keepdims=True) * (1.0 / 16.0) + 1.0
        dinv = lax.rsqrt(total)
        hw = jnp.dot(x_ref[...], w_ref[...],
                     preferred_element_type=jnp.float32,
                     precision=lax.Precision.HIGHEST)
        dinv_ref[...] = dinv
        g_ref[...] = hw * dinv

    return pl.pallas_call(
        body,
        grid=(grid,),
        in_specs=[
            pl.BlockSpec((_NC, _BR, 128), lambda i: (0, i, 0)),
            pl.BlockSpec((_BR, d_in), lambda i: (i, 0)),
            pl.BlockSpec((d_in, hid), lambda i: (0, 0)),
        ],
        out_specs=[
            pl.BlockSpec((_BR, 1), lambda i: (i, 0)),
            pl.BlockSpec((_BR, hid), lambda i: (i, 0)),
        ],
        out_shape=[
            jax.ShapeDtypeStruct((n, 1), jnp.float32),
            jax.ShapeDtypeStruct((n_acc, hid), jnp.float32),
        ],
    )


def _mid_tc(n, n_acc, d, d_next, d_store):
    grid = n // _BR

    def body(agg_ref, g_ref, dinv_ref, b_ref, w_ref, out_ref):
        s = (agg_ref[0] + agg_ref[1] + g_ref[...]) * dinv_ref[...] + b_ref[...]
        h = jnp.maximum(s, 0.0)
        w = w_ref[...]
        if d_store > d_next:
            w = jnp.concatenate(
                [w, jnp.zeros((d, d_store - d_next), jnp.float32)], axis=1)
        out_ref[...] = jnp.dot(h, w,
                               preferred_element_type=jnp.float32,
                               precision=lax.Precision.HIGHEST) * dinv_ref[...]

    return pl.pallas_call(
        body,
        grid=(grid,),
        in_specs=[
            pl.BlockSpec((_NC, _BR, d), lambda i: (0, i, 0)),
            pl.BlockSpec((_BR, d), lambda i: (i, 0)),
            pl.BlockSpec((_BR, 1), lambda i: (i, 0)),
            pl.BlockSpec((1, d), lambda i: (0, 0)),
            pl.BlockSpec((d, d_next), lambda i: (0, 0)),
        ],
        out_specs=pl.BlockSpec((_BR, d_store), lambda i: (i, 0)),
        out_shape=jax.ShapeDtypeStruct((n_acc, d_store), jnp.float32),
    )


def _final_tc(n, d, d_store):
    grid = n // _BR

    def body(agg_ref, g_ref, dinv_ref, b_ref, out_ref):
        agg = agg_ref[0, :, :d] + agg_ref[1, :, :d] + g_ref[:, :d]
        out_ref[...] = agg * dinv_ref[...] + b_ref[...]

    return pl.pallas_call(
        body,
        grid=(grid,),
        in_specs=[
            pl.BlockSpec((_NC, _BR, d_store), lambda i: (0, i, 0)),
            pl.BlockSpec((_BR, d_store), lambda i: (i, 0)),
            pl.BlockSpec((_BR, 1), lambda i: (i, 0)),
            pl.BlockSpec((1, d), lambda i: (0, 0)),
        ],
        out_specs=pl.BlockSpec((_BR, d), lambda i: (i, 0)),
        out_shape=jax.ShapeDtypeStruct((n, d), jnp.float32),
    )


def kernel(x, edge_index, W1, b1, W2, b2, W3, b3):
    n, d_in = x.shape
    hid = W1.shape[1]
    c_out = W3.shape[1]
    e = edge_index.shape[1]

    ew = _NW * _K
    chunks = -(-e // ew)
    e_pad = chunks * ew
    slab = _NS * _K
    n_acc = -(-(n + 1) // slab) * slab

    pad = e_pad - e
    fill = jnp.full((pad,), n, dtype=jnp.int32)
    srcp = jnp.concatenate([edge_index[0], fill]).reshape(_NW, chunks, _K)
    dstp = jnp.concatenate([edge_index[1], fill]).reshape(_NW, chunks, _K)

    c_store = -(-c_out // 128) * 128

    degp = _deg_kernel(n_acc, chunks)(dstp)
    dinv, g1 = _first_tc(n, n_acc, d_in, hid)(degp, x, W1)
    agg1 = _agg_kernel(n_acc, hid, chunks)(g1, srcp, dstp)
    g2 = _mid_tc(n, n_acc, hid, hid, hid)(agg1, g1, dinv, b1.reshape(1, -1), W2)
    agg2 = _agg_kernel(n_acc, hid, chunks)(g2, srcp, dstp)
    g3 = _mid_tc(n, n_acc, hid, c_out, c_store)(agg2, g2, dinv,
                                                b2.reshape(1, -1), W3)
    agg3 = _agg_kernel(n_acc, c_store, chunks)(g3, srcp, dstp)
    return _final_tc(n, c_out, c_store)(agg3, g3, dinv, b3.reshape(1, -1))
